# baseline (speedup 1.0000x reference)
import jax
import jax.numpy as jnp
from jax import lax
from jax.experimental import pallas as pl


def _build_one_hot_t(B, D, H, BB):
    def body(x_ref, o_ref):
        i = pl.program_id(0)
        x = x_ref[:, pl.ds(i * BB, BB)]
        d = lax.broadcasted_iota(jnp.int32, (1, D, 1), 1)
        o_ref[...] = (x[:, None, :] == d).astype(jnp.float32)

    return pl.pallas_call(
        body,
        out_shape=jax.ShapeDtypeStruct((H, D, B), jnp.float32),
        grid=(B // BB,),
        in_specs=[pl.BlockSpec((H, B), lambda i: (0, 0))],
        out_specs=pl.BlockSpec((H, D, BB), lambda i: (0, 0, i)),
    )


def kernel(X_in, ones):
    D = ones.shape[0]
    B, H = X_in.shape
    out_t = _build_one_hot_t(B, D, H, BB=256)(X_in.T)
    return jnp.transpose(out_t, (2, 1, 0))


# speedup vs baseline: 1.0172x; 1.0172x over previous
import jax
import jax.numpy as jnp
from jax import lax
from jax.experimental import pallas as pl


def _build_one_hot_t(B, D, H, BB):
    def body(x_ref, o_ref):
        i = pl.program_id(0)
        x = x_ref[:, pl.ds(i * BB, BB)]
        d = lax.broadcasted_iota(jnp.int32, (1, D, 1), 1)
        o_ref[...] = jnp.where(x[:, None, :] == d, 1.0, 0.0).astype(jnp.float32)

    return pl.pallas_call(
        body,
        out_shape=jax.ShapeDtypeStruct((H, D, B), jnp.float32),
        grid=(B // BB,),
        in_specs=[pl.BlockSpec((H, B), lambda i: (0, 0))],
        out_specs=pl.BlockSpec((H, D, BB), lambda i: (0, 0, i)),
    )


def kernel(X_in, ones):
    D = ones.shape[0]
    B, H = X_in.shape
    out_t = _build_one_hot_t(B, D, H, BB=128)(X_in.T)
    return jnp.transpose(out_t, (2, 1, 0))
